# baseline (speedup 1.0000x reference)
import functools

import jax
import jax.numpy as jnp
from jax.experimental import pallas as pl
from jax.experimental.pallas import tpu as pltpu


def _tile_kernel(ei_ref, ej_ref, li_ref, lj_ref, out_ref, *, inv_d):
    g = jnp.dot(ei_ref[...], ej_ref[...], preferred_element_type=jnp.float32)

    ei = ei_ref[...].astype(jnp.float32)
    ej = ej_ref[...].astype(jnp.float32)
    sqi = jnp.sum(ei * ei, axis=1, keepdims=True)
    sqj = jnp.sum(ej * ej, axis=0, keepdims=True)

    mse = (sqi + sqj - 2.0 * g) * inv_d
    same = li_ref[:, 0:1] == lj_ref[0:1, :]
    val = jnp.where(same, mse, jnp.maximum(1.0 - mse, 0.0))
    part = jnp.sum(val, axis=0, keepdims=True)

    @pl.when(pl.program_id(1) == 0)
    def _init():
        out_ref[...] = jnp.zeros_like(out_ref)

    out_ref[...] += part[None]


def kernel(embeddings, labels):
    B, D = embeddings.shape
    BM = 512
    BN = 512
    ni = B // BM
    nj = B // BN

    eb = embeddings.astype(jnp.bfloat16)
    ebt = eb.T
    labf = labels.astype(jnp.float32)
    lab_col = jnp.broadcast_to(labf[:, None], (B, 8))
    lab_row = jnp.broadcast_to(labf[None, :], (8, B))

    partial = pl.pallas_call(
        functools.partial(_tile_kernel, inv_d=1.0 / D),
        grid=(ni, nj),
        in_specs=[
            pl.BlockSpec((BM, D), lambda i, j: (i, 0)),
            pl.BlockSpec((D, BN), lambda i, j: (0, j)),
            pl.BlockSpec((BM, 8), lambda i, j: (i, 0)),
            pl.BlockSpec((8, BN), lambda i, j: (0, j)),
        ],
        out_specs=pl.BlockSpec((1, 1, BN), lambda i, j: (i, 0, 0)),
        out_shape=jax.ShapeDtypeStruct((ni, 1, BN), jnp.float32),
        compiler_params=pltpu.CompilerParams(
            dimension_semantics=("parallel", "arbitrary"),
        ),
    )(eb, ebt, lab_col, lab_row)

    total = jnp.sum(partial)
    return total / (2.0 * B * (B - 1))
